# baseline (speedup 1.0000x reference)
import functools

import jax
import jax.numpy as jnp
from jax import lax
from jax.experimental import pallas as pl
from jax.experimental.pallas import tpu as pltpu
from jax.experimental.pallas import tpu_sc as plsc

_B = 1024
_K = 100000
_SCB = 256
_TCB = _B - _SCB
_BM = 64
_NW = 32
_RPW = _SCB // _NW
_W = 2000
_NC = _K // _W


def _tc_body(s_ref, v_ref, o_ref):
    o_ref[...] = jnp.sum(s_ref[...] * v_ref[...], axis=1, keepdims=True)


_tc_matvec = pl.pallas_call(
    _tc_body,
    grid=(_TCB // _BM,),
    in_specs=[
        pl.BlockSpec((_BM, _K), lambda b: (b, 0)),
        pl.BlockSpec((1, _K), lambda b: (0, 0)),
    ],
    out_specs=pl.BlockSpec((_BM, 1), lambda b: (b, 0)),
    out_shape=jax.ShapeDtypeStruct((_TCB, 1), jnp.float32),
)


def _sc_body(state_hbm, values_hbm, out_hbm, sbuf, vbuf, outv,
             ssem0, ssem1, vsem0, vsem1):
    wid = lax.axis_index("s") * 2 + lax.axis_index("c")
    row0 = _TCB + wid * _RPW
    lanes = lax.iota(jnp.int32, 16)
    ssems = (ssem0, ssem1)
    vsems = (vsem0, vsem1)

    def start(c, slot):
        pltpu.async_copy(
            state_hbm.at[pl.ds(row0, _RPW), pl.ds(c * _W, _W)],
            sbuf.at[slot], ssems[slot])
        pltpu.async_copy(
            values_hbm.at[pl.ds(c * _W, _W)], vbuf.at[slot], vsems[slot])

    def wait(c, slot):
        pltpu.make_async_copy(
            state_hbm.at[pl.ds(row0, _RPW), pl.ds(c * _W, _W)],
            sbuf.at[slot], ssems[slot]).wait()
        pltpu.make_async_copy(
            values_hbm.at[pl.ds(c * _W, _W)], vbuf.at[slot],
            vsems[slot]).wait()

    def compute(slot, accs):
        def j_body(j, accs):
            v = vbuf[slot, pl.ds(j * 16, 16)]
            return tuple(
                accs[i] + sbuf[slot, i, pl.ds(j * 16, 16)] * v
                for i in range(_RPW))
        return plsc.parallel_loop(
            0, _W // 16, 1, unroll=2, carry=accs)(j_body)

    start(0, 0)
    start(1, 1)

    def pair_body(p, accs):
        for b in range(2):
            c = 2 * p + b
            wait(c, b)
            accs = compute(b, accs)

            @pl.when(c + 2 < _NC)
            def _(c=c, b=b):
                start(c + 2, b)
        return accs

    accs0 = tuple(jnp.zeros((16,), jnp.float32) for _ in range(_RPW))
    accs = lax.fori_loop(0, _NC // 2, pair_body, accs0)

    out_vec = jnp.zeros((16,), jnp.float32)
    for i in range(_RPW):
        out_vec = jnp.where(lanes == i, jnp.sum(accs[i]), out_vec)
    outv[...] = out_vec

    pltpu.sync_copy(outv.at[pl.ds(0, _RPW)],
                    out_hbm.at[pl.ds(wid * _RPW, _RPW)])


_sc_matvec = functools.partial(
    pl.kernel,
    out_type=jax.ShapeDtypeStruct((_SCB,), jnp.float32),
    mesh=plsc.VectorSubcoreMesh(core_axis_name="c", subcore_axis_name="s"),
    scratch_types=[
        pltpu.VMEM((2, _RPW, _W), jnp.float32),
        pltpu.VMEM((2, _W), jnp.float32),
        pltpu.VMEM((16,), jnp.float32),
        pltpu.SemaphoreType.DMA,
        pltpu.SemaphoreType.DMA,
        pltpu.SemaphoreType.DMA,
        pltpu.SemaphoreType.DMA,
    ],
    compiler_params=pltpu.CompilerParams(
        use_tc_tiling_on_sc=False, needs_layout_passes=False),
)(_sc_body)


def kernel(state, values):
    out_tc = _tc_matvec(state, values.reshape(1, _K))
    out_sc = _sc_matvec(state, values.reshape(_K))
    return jnp.concatenate([out_tc, out_sc.reshape(_SCB, 1)], axis=0)


# speedup vs baseline: 2.2766x; 2.2766x over previous
import jax
import jax.numpy as jnp
from jax.experimental import pallas as pl
from jax.experimental.pallas import tpu as pltpu

_B = 1024
_K = 100000
_BM = 64
_NM = _B // _BM


def _body(s_ref, v_ref, o_ref):
    o_ref[...] = jnp.sum(s_ref[...] * v_ref[...], axis=1, keepdims=True)


_matvec = pl.pallas_call(
    _body,
    grid=(_NM,),
    in_specs=[
        pl.BlockSpec((_BM, _K), lambda b: (b, 0)),
        pl.BlockSpec((1, _K), lambda b: (0, 0)),
    ],
    out_specs=pl.BlockSpec((_BM, 1), lambda b: (b, 0)),
    out_shape=jax.ShapeDtypeStruct((_B, 1), jnp.float32),
    compiler_params=pltpu.CompilerParams(
        dimension_semantics=("parallel",)),
)


def kernel(state, values):
    return _matvec(state, values.reshape(1, _K))
